# baseline (speedup 1.0000x reference)
import functools
import math

import jax
import jax.numpy as jnp
from jax import lax
from jax.experimental import pallas as pl
from jax.experimental.pallas import tpu as pltpu
from jax.experimental.pallas import tpu_sc as plsc

_GR = 128


def _build_emb(n_idx, d, scale):
    nc, ns = 2, 16
    n_workers = nc * ns
    rows_per_w = n_idx // n_workers
    n_steps = rows_per_w // _GR
    mesh = plsc.VectorSubcoreMesh(core_axis_name="c", subcore_axis_name="s")

    @functools.partial(
        pl.kernel,
        mesh=mesh,
        out_type=jax.ShapeDtypeStruct((n_idx, d), jnp.float32),
        scratch_types=[
            pltpu.VMEM((n_steps, _GR), jnp.int32),
            pltpu.VMEM((_GR, d), jnp.float32),
            pltpu.SemaphoreType.DMA,
        ],
        compiler_params=pltpu.CompilerParams(use_tc_tiling_on_sc=False),
    )
    def emb(idx_hbm, table_hbm, out_hbm, idx_v, rows_v, sem):
        wid = lax.axis_index("s") * nc + lax.axis_index("c")
        base_step = wid * n_steps
        pltpu.sync_copy(idx_hbm.at[pl.ds(base_step, n_steps)], idx_v)

        def step(j, carry):
            pltpu.async_copy(table_hbm.at[idx_v.at[j]], rows_v, sem).wait()

            def scale_row(r, c2):
                for k in range(d // 16):
                    sl = pl.ds(k * 16, 16)
                    rows_v[r, sl] = rows_v[r, sl] * scale
                return c2

            lax.fori_loop(0, _GR, scale_row, 0)
            pltpu.sync_copy(
                rows_v, out_hbm.at[pl.ds((base_step + j) * _GR, _GR)]
            )
            return carry

        lax.fori_loop(0, n_steps, step, 0)

    return emb


def kernel(tokens, table):
    b, l = tokens.shape
    v, d = table.shape
    n = b * l
    scale = math.sqrt(d)
    idx = tokens.reshape(n // _GR, _GR)
    out = _build_emb(n, d, scale)(idx, table)
    return out.reshape(b, l, d)


# speedup vs baseline: 1.2080x; 1.2080x over previous
import functools
import math

import jax
import jax.numpy as jnp
from jax import lax
from jax.experimental import pallas as pl
from jax.experimental.pallas import tpu as pltpu
from jax.experimental.pallas import tpu_sc as plsc

_GR = 128
_NBUF = 4


def _build_emb(n_idx, d, scale):
    nc, ns = 2, 16
    n_workers = nc * ns
    rows_per_w = n_idx // n_workers
    n_steps = rows_per_w // _GR
    mesh = plsc.VectorSubcoreMesh(core_axis_name="c", subcore_axis_name="s")

    @functools.partial(
        pl.kernel,
        mesh=mesh,
        out_type=jax.ShapeDtypeStruct((n_idx, d), jnp.float32),
        scratch_types=[
            pltpu.VMEM((n_steps, _GR), jnp.int32),
            pltpu.VMEM((_NBUF, _GR, d), jnp.float32),
            pltpu.VMEM((_NBUF, _GR, d), jnp.float32),
            pltpu.SemaphoreType.DMA((_NBUF,)),
            pltpu.SemaphoreType.DMA((_NBUF,)),
        ],
        compiler_params=pltpu.CompilerParams(use_tc_tiling_on_sc=False),
    )
    def emb(idx_hbm, table_hbm, out_hbm, idx_v, rows_g, rows_s, gsem, ssem):
        wid = lax.axis_index("s") * nc + lax.axis_index("c")
        base_step = wid * n_steps
        pltpu.sync_copy(idx_hbm.at[pl.ds(base_step, n_steps)], idx_v)

        def start_gather(step, b):
            pltpu.async_copy(
                table_hbm.at[idx_v.at[step]], rows_g.at[b], gsem.at[b]
            )

        for b in range(_NBUF):
            start_gather(b, b)

        def group(g, carry):
            for b in range(_NBUF):
                step = g * _NBUF + b
                pltpu.make_async_copy(
                    table_hbm.at[idx_v.at[step]], rows_g.at[b], gsem.at[b]
                ).wait()
                @pl.when(step >= _NBUF)
                def _():
                    pltpu.make_async_copy(
                        rows_s.at[b],
                        out_hbm.at[pl.ds(0, _GR)],
                        ssem.at[b],
                    ).wait()

                def scale_row(r, c2):
                    for k in range(d // 16):
                        sl = pl.ds(k * 16, 16)
                        rows_s[b, r, sl] = rows_g[b, r, sl] * scale
                    return c2

                lax.fori_loop(0, _GR, scale_row, 0)
                pltpu.async_copy(
                    rows_s.at[b],
                    out_hbm.at[pl.ds((base_step + step) * _GR, _GR)],
                    ssem.at[b],
                )

                @pl.when(step + _NBUF < n_steps)
                def _():
                    start_gather(step + _NBUF, b)

            return carry

        lax.fori_loop(0, n_steps // _NBUF, group, 0)

        for b in range(_NBUF):
            pltpu.make_async_copy(
                rows_s.at[b], out_hbm.at[pl.ds(0, _GR)], ssem.at[b]
            ).wait()

    return emb


def kernel(tokens, table):
    b, l = tokens.shape
    v, d = table.shape
    n = b * l
    scale = math.sqrt(d)
    idx = tokens.reshape(n // _GR, _GR)
    out = _build_emb(n, d, scale)(idx, table)
    return out.reshape(b, l, d)
